# baseline (speedup 1.0000x reference)
import functools

import jax
import jax.numpy as jnp
from jax import lax
from jax.experimental import pallas as pl
from jax.experimental.pallas import tpu as pltpu
from jax.experimental.pallas import tpu_sc as plsc

_H = 32
_W = 32
_HW = _H * _W
_DH = 384
_D = 768
_L = 16
_NC = 2
_NS = 16
_RPT = _HW // _NS
_HPT = _RPT // _W
_NA = 20
_WINDOW = 8


def _sc_body(batch, row_hbm, col_hbm, idx_hbm, idxr_hbm, out_hbm,
             idx_v, idxr_v, colrows_v, rowtwo_v, buf_v, shared, gsem, osem, ssem):
    cid = lax.axis_index("c")
    sid = lax.axis_index("s")
    bpc = batch // _NC
    nb = bpc - _NA
    pltpu.sync_copy(idx_hbm, idx_v)
    pltpu.sync_copy(idxr_hbm, idxr_v)
    cgather = pltpu.make_async_copy(col_hbm.at[idx_v], colrows_v, gsem)
    rgather = pltpu.make_async_copy(
        row_hbm.at[idxr_v.at[pl.ds(8 * sid, _HPT)]], rowtwo_v, gsem)
    cgather.start()
    rgather.start()
    cgather.wait()
    rgather.wait()

    nk = _DH // _L
    row_regs = [[rowtwo_v[h, pl.ds(_L * k, _L)] for k in range(nk)]
                for h in range(_HPT)]
    for r in range(_RPT):
        h, w = r // _W, r % _W
        for k in range(nk):
            buf_v[r, pl.ds(_L * k, _L)] = colrows_v[w, pl.ds(_L * k, _L)]
        for k in range(nk):
            buf_v[r, pl.ds(_DH + _L * k, _L)] = row_regs[h][k]

    pltpu.sync_copy(buf_v, shared.at[pl.ds(_RPT * sid, _RPT), :])
    plsc.subcore_barrier()

    spmem_copies = [
        (j, pltpu.make_async_copy(shared, out_hbm.at[bpc * cid + _NA + j], ssem))
        for j in range(nb)
    ]
    for j, cp in spmem_copies:
        @pl.when(jnp.equal(jnp.int32(j), sid))
        def _start(cp=cp):
            cp.start()

    copies = [
        pltpu.make_async_copy(
            buf_v, out_hbm.at[bpc * cid + b, pl.ds(_RPT * sid, _RPT), :], osem)
        for b in range(_NA)
    ]
    for b in range(_NA):
        copies[b].start()
        if b >= _WINDOW:
            copies[b - _WINDOW].wait()
    for b in range(max(_NA - _WINDOW, 0), _NA):
        copies[b].wait()

    for j, cp in spmem_copies:
        @pl.when(jnp.equal(jnp.int32(j), sid))
        def _wait(cp=cp):
            cp.wait()


def kernel(x, height, width, row_embed, col_embed):
    batch = x.shape[0]
    zero = (jnp.asarray(height, jnp.int32) - _H) + (jnp.asarray(width, jnp.int32) - _W)
    idx = jnp.arange(_W, dtype=jnp.int32) + zero
    hpairs = (jnp.arange(_NS * 8, dtype=jnp.int32) // 8) * _HPT
    lane = jnp.arange(_NS * 8, dtype=jnp.int32) % 8
    idxr = hpairs + jnp.minimum(lane, _HPT - 1) + zero
    mesh = plsc.VectorSubcoreMesh(core_axis_name="c", subcore_axis_name="s")
    k = functools.partial(
        pl.kernel,
        mesh=mesh,
        out_type=jax.ShapeDtypeStruct((batch, _HW, _D), jnp.float32),
        scratch_types=[
            pltpu.VMEM((_W,), jnp.int32),
            pltpu.VMEM((_NS * 8,), jnp.int32),
            pltpu.VMEM((_W, _DH), jnp.float32),
            pltpu.VMEM((_HPT, _DH), jnp.float32),
            pltpu.VMEM((_RPT, _D), jnp.float32),
            pltpu.VMEM_SHARED((_HW, _D), jnp.float32),
            pltpu.SemaphoreType.DMA,
            pltpu.SemaphoreType.DMA,
            pltpu.SemaphoreType.DMA,
        ],
    )(functools.partial(_sc_body, batch))
    return k(row_embed, col_embed, idx, idxr)


# speedup vs baseline: 1.0741x; 1.0741x over previous
import functools

import jax
import jax.numpy as jnp
from jax import lax
from jax.experimental import pallas as pl
from jax.experimental.pallas import tpu as pltpu
from jax.experimental.pallas import tpu_sc as plsc

_H = 32
_W = 32
_HW = _H * _W
_DH = 384
_D = 768
_L = 16
_NC = 2
_NS = 16
_WINDOW = 8


def _sc_body(batch, row_hbm, col_hbm, idx_hbm, idxpad_hbm, out_hbm,
             idx_v, idxpad_v, colrows_v, rowone_v, buf_v, gsem, osem):
    wid = lax.axis_index("s") * _NC + lax.axis_index("c")
    pltpu.sync_copy(idx_hbm, idx_v)
    pltpu.sync_copy(idxpad_hbm, idxpad_v)
    cgather = pltpu.make_async_copy(col_hbm.at[idx_v], colrows_v, gsem)
    rgather = pltpu.make_async_copy(
        row_hbm.at[idxpad_v.at[pl.ds(8 * wid, 1)]], rowone_v, gsem)
    cgather.start()
    rgather.start()
    cgather.wait()
    rgather.wait()

    nk = _DH // _L
    row_regs = [rowone_v[0, pl.ds(_L * k, _L)] for k in range(nk)]
    for w in range(_W):
        for k in range(nk):
            buf_v[w, pl.ds(_L * k, _L)] = colrows_v[w, pl.ds(_L * k, _L)]
        for k in range(nk):
            buf_v[w, pl.ds(_DH + _L * k, _L)] = row_regs[k]

    copies = [
        pltpu.make_async_copy(buf_v, out_hbm.at[b, pl.ds(_H * wid, _H), :], osem)
        for b in range(batch)
    ]
    for b in range(batch):
        copies[b].start()
        if b >= _WINDOW:
            copies[b - _WINDOW].wait()
    for b in range(max(batch - _WINDOW, 0), batch):
        copies[b].wait()


def kernel(x, height, width, row_embed, col_embed):
    batch = x.shape[0]
    zero = (jnp.asarray(height, jnp.int32) - _H) + (jnp.asarray(width, jnp.int32) - _W)
    idx = jnp.arange(_W, dtype=jnp.int32) + zero
    idxpad = jnp.repeat(idx, 8)
    mesh = plsc.VectorSubcoreMesh(core_axis_name="c", subcore_axis_name="s")
    k = functools.partial(
        pl.kernel,
        mesh=mesh,
        out_type=jax.ShapeDtypeStruct((batch, _HW, _D), jnp.float32),
        scratch_types=[
            pltpu.VMEM((_W,), jnp.int32),
            pltpu.VMEM((_W * 8,), jnp.int32),
            pltpu.VMEM((_W, _DH), jnp.float32),
            pltpu.VMEM((1, _DH), jnp.float32),
            pltpu.VMEM((_W, _D), jnp.float32),
            pltpu.SemaphoreType.DMA,
            pltpu.SemaphoreType.DMA,
        ],
    )(functools.partial(_sc_body, batch))
    return k(row_embed, col_embed, idx, idxpad)


# speedup vs baseline: 1.0778x; 1.0034x over previous
import functools

import jax
import jax.numpy as jnp
from jax import lax
from jax.experimental import pallas as pl
from jax.experimental.pallas import tpu as pltpu
from jax.experimental.pallas import tpu_sc as plsc

_H = 32
_W = 32
_HW = _H * _W
_DH = 384
_D = 768
_L = 16
_NC = 2
_NS = 16
_WINDOW = 8


def _sc_body(batch, row_hbm, col_hbm, idx_hbm, idxpad_hbm, out_hbm,
             idx_v, idxpad_v, colrows_v, rowone_v, buf_v, gsem, osem):
    wid = lax.axis_index("s") * _NC + lax.axis_index("c")
    pltpu.sync_copy(idx_hbm, idx_v)
    pltpu.sync_copy(idxpad_hbm, idxpad_v)
    cgather = pltpu.make_async_copy(col_hbm.at[idx_v], colrows_v, gsem)
    rgather = pltpu.make_async_copy(
        row_hbm.at[idxpad_v.at[pl.ds(8 * wid, 1)]], rowone_v, gsem)
    cgather.start()
    rgather.start()
    cgather.wait()
    rgather.wait()

    nk = _DH // _L
    row_regs = [rowone_v[0, pl.ds(_L * k, _L)] for k in range(nk)]
    for w in range(_W):
        for k in range(nk):
            buf_v[w, pl.ds(_L * k, _L)] = colrows_v[w, pl.ds(_L * k, _L)]
        for k in range(nk):
            buf_v[w, pl.ds(_DH + _L * k, _L)] = row_regs[k]

    def _issue(b, carry):
        pltpu.make_async_copy(
            buf_v, out_hbm.at[b, pl.ds(_H * wid, _H), :], osem).start()

        @pl.when(b >= _WINDOW)
        def _retire():
            pltpu.make_async_copy(
                buf_v, out_hbm.at[0, pl.ds(_H * wid, _H), :], osem).wait()

        return carry

    lax.fori_loop(0, batch, _issue, 0)
    for _ in range(min(_WINDOW, batch)):
        pltpu.make_async_copy(
            buf_v, out_hbm.at[0, pl.ds(_H * wid, _H), :], osem).wait()


def kernel(x, height, width, row_embed, col_embed):
    batch = x.shape[0]
    zero = (jnp.asarray(height, jnp.int32) - _H) + (jnp.asarray(width, jnp.int32) - _W)
    idx = jnp.arange(_W, dtype=jnp.int32) + zero
    idxpad = jnp.repeat(idx, 8)
    mesh = plsc.VectorSubcoreMesh(core_axis_name="c", subcore_axis_name="s")
    k = functools.partial(
        pl.kernel,
        mesh=mesh,
        out_type=jax.ShapeDtypeStruct((batch, _HW, _D), jnp.float32),
        scratch_types=[
            pltpu.VMEM((_W,), jnp.int32),
            pltpu.VMEM((_W * 8,), jnp.int32),
            pltpu.VMEM((_W, _DH), jnp.float32),
            pltpu.VMEM((1, _DH), jnp.float32),
            pltpu.VMEM((_W, _D), jnp.float32),
            pltpu.SemaphoreType.DMA,
            pltpu.SemaphoreType.DMA,
        ],
    )(functools.partial(_sc_body, batch))
    return k(row_embed, col_embed, idx, idxpad)
